# baseline (speedup 1.0000x reference)
import functools

import jax
import jax.numpy as jnp
from jax import lax
from jax.experimental import pallas as pl
from jax.experimental.pallas import tpu as pltpu
from jax.experimental.pallas import tpu_sc as plsc

L = 16
NC = 2
NS = 16
NW = NC * NS
CHUNK = 128


def _mf_body(uidx_hbm, iidx_hbm, usr_nid_hbm, itm_nid_hbm,
             u1_hbm, i1_hbm, out_hbm,
             uidx_v, iidx_v, cu_v, ci_v, fu_v, fi_v,
             urows_v, irows_v, out_v,
             sem_idx, sem_rows):
    V = u1_hbm.shape[0] // L
    wid = lax.axis_index("s") * NC + lax.axis_index("c")
    nchunk = uidx_v.shape[0]
    epw = nchunk * CHUNK
    base_row = wid * nchunk

    pltpu.sync_copy(uidx_hbm.at[pl.ds(base_row, nchunk)], uidx_v)
    pltpu.sync_copy(iidx_hbm.at[pl.ds(base_row, nchunk)], iidx_v)

    cps = []
    for c in range(nchunk):
        cps.append(pltpu.async_copy(
            usr_nid_hbm.at[uidx_v.at[c]], cu_v.at[c], sem_idx))
        cps.append(pltpu.async_copy(
            itm_nid_hbm.at[iidx_v.at[c]], ci_v.at[c], sem_idx))
    for cp in cps:
        cp.wait()

    def build(s, carry):
        rowu = cu_v[s // (CHUNK // L), pl.ds((s % (CHUNK // L)) * L, L)]
        rowi = ci_v[s // (CHUNK // L), pl.ds((s % (CHUNK // L)) * L, L)]
        for d in range(L):
            dv = jnp.full((L,), d * V, jnp.int32)
            fu_v[d * nchunk + s // (CHUNK // L),
                 pl.ds((s % (CHUNK // L)) * L, L)] = rowu + dv
            fi_v[d * nchunk + s // (CHUNK // L),
                 pl.ds((s % (CHUNK // L)) * L, L)] = rowi + dv
        return carry

    lax.fori_loop(0, epw // L, build, 0)

    cps = []
    for d in range(L):
        for c in range(nchunk):
            cps.append(pltpu.async_copy(
                u1_hbm.at[fu_v.at[d * nchunk + c]],
                urows_v.at[d, pl.ds(c * CHUNK, CHUNK)], sem_rows))
            cps.append(pltpu.async_copy(
                i1_hbm.at[fi_v.at[d * nchunk + c]],
                irows_v.at[d, pl.ds(c * CHUNK, CHUNK)], sem_rows))
    for cp in cps:
        cp.wait()

    def group(g, carry):
        b = g * L
        acc = urows_v[0, pl.ds(b, L)] * irows_v[0, pl.ds(b, L)]
        for d in range(1, L):
            acc = acc + urows_v[d, pl.ds(b, L)] * irows_v[d, pl.ds(b, L)]
        out_v[pl.ds(b, L)] = acc
        return carry

    lax.fori_loop(0, epw // L, group, 0)

    pltpu.sync_copy(out_v, out_hbm.at[pl.ds(wid * epw, epw)])


def kernel(usr_n_id, itm_n_id, edge_label_index, usr_table, itm_table):
    B = usr_n_id.shape[0]
    epw = B // NW
    nchunk = epw // CHUNK

    usr_idx = edge_label_index[0].astype(jnp.int32).reshape(B // CHUNK, CHUNK)
    itm_idx = edge_label_index[1].astype(jnp.int32).reshape(B // CHUNK, CHUNK)
    usr_n_id = usr_n_id.astype(jnp.int32)
    itm_n_id = itm_n_id.astype(jnp.int32)

    u1 = usr_table.T.reshape(-1)
    i1 = itm_table.T.reshape(-1)

    mesh = plsc.VectorSubcoreMesh(core_axis_name="c", subcore_axis_name="s")
    f = functools.partial(
        pl.kernel,
        mesh=mesh,
        out_type=jax.ShapeDtypeStruct((B,), jnp.float32),
        scratch_types=[
            pltpu.VMEM((nchunk, CHUNK), jnp.int32),
            pltpu.VMEM((nchunk, CHUNK), jnp.int32),
            pltpu.VMEM((nchunk, CHUNK), jnp.int32),
            pltpu.VMEM((nchunk, CHUNK), jnp.int32),
            pltpu.VMEM((L * nchunk, CHUNK), jnp.int32),
            pltpu.VMEM((L * nchunk, CHUNK), jnp.int32),
            pltpu.VMEM((L, epw), jnp.float32),
            pltpu.VMEM((L, epw), jnp.float32),
            pltpu.VMEM((epw,), jnp.float32),
            pltpu.SemaphoreType.DMA,
            pltpu.SemaphoreType.DMA,
        ],
    )(_mf_body)
    return f(usr_idx, itm_idx, usr_n_id, itm_n_id, u1, i1)
